# Initial kernel scaffold; baseline (speedup 1.0000x reference)
import jax
import jax.numpy as jnp
from jax.experimental import pallas as pl


def kernel(x, edge_index, W1, b1):
    raise NotImplementedError("write your pallas kernel here")


# speedup vs baseline: 31.4444x; 31.4444x over previous
import functools

import jax
import jax.numpy as jnp
from jax import lax
from jax.experimental import pallas as pl
from jax.experimental.pallas import tpu as pltpu
from jax.experimental.pallas import tpu_sc as plsc

_SCALE = 1.8
_L = 16
_NC = 2
_NS = 16
_NW = _NC * _NS
_K = 128
_BLK = 1000


def _sc_mesh():
    return plsc.VectorSubcoreMesh(
        core_axis_name="c", subcore_axis_name="s",
        num_cores=_NC, num_subcores=_NS)


@functools.lru_cache(maxsize=None)
def _make_degree_fn(n_pad, ch):
    zrows = n_pad // _NS

    @functools.partial(
        pl.kernel,
        out_type=jax.ShapeDtypeStruct((_NC * n_pad,), jnp.float32),
        mesh=_sc_mesh(),
        scratch_types=[
            pltpu.VMEM((ch, _K), jnp.int32),
            pltpu.VMEM((_K,), jnp.float32),
            pltpu.VMEM((zrows,), jnp.float32),
            pltpu.VMEM_SHARED((n_pad,), jnp.float32),
            pltpu.SemaphoreType.DMA,
        ],
    )
    def deg_fn(dst_hbm, out_hbm, idx_v, ones_v, zero_v, hist_sp, sem):
        del sem
        c = lax.axis_index("c")
        s = lax.axis_index("s")
        w = s * _NC + c
        zero16 = jnp.zeros((_L,), jnp.float32)
        one16 = jnp.ones((_L,), jnp.float32)
        for i in range(zrows // _L):
            zero_v[pl.ds(i * _L, _L)] = zero16
        for i in range(_K // _L):
            ones_v[pl.ds(i * _L, _L)] = one16
        pltpu.sync_copy(zero_v, hist_sp.at[pl.ds(s * zrows, zrows)])
        plsc.subcore_barrier()
        pltpu.sync_copy(dst_hbm.at[w], idx_v)

        def body(j, carry):
            pltpu.sync_copy(ones_v, hist_sp.at[idx_v.at[j]], add=True)
            return carry

        lax.fori_loop(0, ch, body, 0)
        plsc.subcore_barrier()
        pltpu.sync_copy(hist_sp.at[pl.ds(s * zrows, zrows)],
                        out_hbm.at[pl.ds(c * n_pad + s * zrows, zrows)])

    return deg_fn


@functools.lru_cache(maxsize=None)
def _make_scatter_fn(n_pad, ch, d):
    zrows = n_pad // _NS

    @functools.partial(
        pl.kernel,
        out_type=jax.ShapeDtypeStruct((_NC, n_pad, d), jnp.float32),
        mesh=_sc_mesh(),
        scratch_types=[
            pltpu.VMEM((ch, _K), jnp.int32),
            pltpu.VMEM((ch, _K), jnp.int32),
            pltpu.VMEM((_K, d), jnp.float32),
            pltpu.VMEM((_L, d), jnp.float32),
            pltpu.VMEM_SHARED((n_pad, d), jnp.float32),
            pltpu.SemaphoreType.DMA,
        ],
    )
    def scat_fn(g_hbm, src_hbm, dst_hbm, out_hbm,
                src_v, dst_v, buf, zbuf, acc_sp, sem):
        c = lax.axis_index("c")
        s = lax.axis_index("s")
        w = s * _NC + c
        zero16 = jnp.zeros((_L,), jnp.float32)
        for i in range(_L):
            for j in range(d // _L):
                zbuf[i, pl.ds(j * _L, _L)] = zero16
        for r in range(zrows // _L):
            pltpu.sync_copy(zbuf, acc_sp.at[pl.ds(s * zrows + r * _L, _L)])
        plsc.subcore_barrier()
        pltpu.sync_copy(src_hbm.at[w], src_v)
        pltpu.sync_copy(dst_hbm.at[w], dst_v)

        def body(j, carry):
            pltpu.async_copy(g_hbm.at[src_v.at[j]], buf, sem).wait()
            pltpu.sync_copy(buf, acc_sp.at[dst_v.at[j]], add=True)
            return carry

        lax.fori_loop(0, ch, body, 0)
        plsc.subcore_barrier()
        pltpu.sync_copy(acc_sp.at[pl.ds(s * zrows, zrows)],
                        out_hbm.at[c, pl.ds(s * zrows, zrows)])

    return scat_fn


def _dense_body(x_ref, w_ref, b_ref, hist_ref, g_ref):
    h = jnp.dot(x_ref[...], w_ref[...], preferred_element_type=jnp.float32)
    h = h + b_ref[...]
    nrm = jnp.sqrt(jnp.sum(h * h, axis=1, keepdims=True))
    deg = hist_ref[:, 0:1] + hist_ref[:, 1:2] + 1.0
    scale = lax.rsqrt(deg) * (_SCALE / jnp.maximum(nrm, 1e-12))
    g_ref[...] = h * scale


def _combine_body(tmp_ref, g_ref, hist_ref, o_ref):
    t = tmp_ref[0] + tmp_ref[1] + g_ref[...]
    deg = hist_ref[:, 0:1] + hist_ref[:, 1:2] + 1.0
    o_ref[...] = t * lax.rsqrt(deg)


def kernel(x, edge_index, W1, b1):
    n, d = x.shape
    e = edge_index.shape[1]
    ch = -(-e // (_NW * _K))
    e_pad = _NW * ch * _K
    pad = e_pad - e
    spread = 240
    n_pad = ((n + spread + 255) // 256) * 256

    src = edge_index[0]
    dst = edge_index[1]
    pidx = jnp.arange(pad, dtype=jnp.int32)
    src2 = jnp.concatenate([src, pidx % 128]).reshape(_NW, ch, _K)
    dst2 = jnp.concatenate([dst, n + pidx % spread]).reshape(_NW, ch, _K)

    hist = _make_degree_fn(n_pad, ch)(dst2).reshape(_NC, n_pad)
    hist_t = hist.T

    grid = n // _BLK
    g = pl.pallas_call(
        _dense_body,
        grid=(grid,),
        in_specs=[
            pl.BlockSpec((_BLK, d), lambda i: (i, 0)),
            pl.BlockSpec((d, d), lambda i: (0, 0)),
            pl.BlockSpec((1, d), lambda i: (0, 0)),
            pl.BlockSpec((_BLK, _NC), lambda i: (i, 0)),
        ],
        out_specs=pl.BlockSpec((_BLK, d), lambda i: (i, 0)),
        out_shape=jax.ShapeDtypeStruct((n, d), jnp.float32),
    )(x, W1, b1.reshape(1, d), hist_t[:n])

    tmp = _make_scatter_fn(n_pad, ch, d)(g, src2, dst2)

    out = pl.pallas_call(
        _combine_body,
        grid=(grid,),
        in_specs=[
            pl.BlockSpec((_NC, _BLK, d), lambda i: (0, i, 0)),
            pl.BlockSpec((_BLK, d), lambda i: (i, 0)),
            pl.BlockSpec((_BLK, _NC), lambda i: (i, 0)),
        ],
        out_specs=pl.BlockSpec((_BLK, d), lambda i: (i, 0)),
        out_shape=jax.ShapeDtypeStruct((n, d), jnp.float32),
    )(tmp, g, hist_t[:n])
    return out


# speedup vs baseline: 41.4431x; 1.3180x over previous
import functools

import jax
import jax.numpy as jnp
from jax import lax
from jax.experimental import pallas as pl
from jax.experimental.pallas import tpu as pltpu
from jax.experimental.pallas import tpu_sc as plsc

_SCALE = 1.8
_L = 16
_NC = 2
_NS = 16
_NW = _NC * _NS
_K = 128
_IB = 8
_BLK = 1000


def _sc_mesh():
    return plsc.VectorSubcoreMesh(
        core_axis_name="c", subcore_axis_name="s",
        num_cores=_NC, num_subcores=_NS)


@functools.lru_cache(maxsize=None)
def _make_degree_fn(n_pad, ch):
    zrows = n_pad // _NS

    @functools.partial(
        pl.kernel,
        out_type=jax.ShapeDtypeStruct((_NC * n_pad,), jnp.float32),
        mesh=_sc_mesh(),
        scratch_types=[
            pltpu.VMEM((ch, _K), jnp.int32),
            pltpu.VMEM((_K,), jnp.float32),
            pltpu.VMEM((zrows,), jnp.float32),
            pltpu.VMEM_SHARED((n_pad,), jnp.float32),
            pltpu.SemaphoreType.DMA,
        ],
    )
    def deg_fn(dst_hbm, out_hbm, idx_v, ones_v, zero_v, hist_sp, sem):
        del sem
        c = lax.axis_index("c")
        s = lax.axis_index("s")
        w = s * _NC + c
        zero16 = jnp.zeros((_L,), jnp.float32)
        one16 = jnp.ones((_L,), jnp.float32)
        for i in range(zrows // _L):
            zero_v[pl.ds(i * _L, _L)] = zero16
        for i in range(_K // _L):
            ones_v[pl.ds(i * _L, _L)] = one16
        pltpu.sync_copy(zero_v, hist_sp.at[pl.ds(s * zrows, zrows)])
        plsc.subcore_barrier()
        pltpu.sync_copy(dst_hbm.at[w], idx_v)

        def body(j, carry):
            pltpu.sync_copy(ones_v, hist_sp.at[idx_v.at[j]], add=True)
            return carry

        lax.fori_loop(0, ch, body, 0)
        plsc.subcore_barrier()
        pltpu.sync_copy(hist_sp.at[pl.ds(s * zrows, zrows)],
                        out_hbm.at[pl.ds(c * n_pad + s * zrows, zrows)])

    return deg_fn


@functools.lru_cache(maxsize=None)
def _make_scatter_fn(n_pad, ch, d):
    zrows = n_pad // _NS
    nb = ch // _IB
    zr = 8

    @functools.partial(
        pl.kernel,
        out_type=jax.ShapeDtypeStruct((_NC, n_pad, d), jnp.float32),
        mesh=_sc_mesh(),
        scratch_types=[
            pltpu.VMEM((ch, _K), jnp.int32),
            pltpu.VMEM((_IB, _K), jnp.int32),
            pltpu.VMEM((_K, d), jnp.float32),
            pltpu.VMEM((_K, d), jnp.float32),
            pltpu.VMEM((zr, d), jnp.float32),
            pltpu.VMEM_SHARED((n_pad, d), jnp.float32),
            pltpu.SemaphoreType.DMA,
            pltpu.SemaphoreType.DMA,
        ],
    )
    def scat_fn(g_hbm, src_hbm, dst_hbm, out_hbm,
                src_v, dst_v, buf0, buf1, zbuf, acc_sp, sem0, sem1):
        c = lax.axis_index("c")
        s = lax.axis_index("s")
        w = s * _NC + c
        zero16 = jnp.zeros((_L,), jnp.float32)
        for i in range(zr):
            for j in range(d // _L):
                zbuf[i, pl.ds(j * _L, _L)] = zero16
        pltpu.sync_copy(src_hbm.at[w], src_v)
        for r in range(zrows // zr):
            pltpu.sync_copy(zbuf, acc_sp.at[pl.ds(s * zrows + r * zr, zr)])
        plsc.subcore_barrier()

        pltpu.async_copy(g_hbm.at[src_v.at[0]], buf0, sem0)

        def blk(b, carry):
            pltpu.sync_copy(dst_hbm.at[w, b], dst_v)
            for t in range(_IB):
                j = b * _IB + t
                cur, csem = (buf0, sem0) if t % 2 == 0 else (buf1, sem1)
                nxt, nsem = (buf1, sem1) if t % 2 == 0 else (buf0, sem0)

                @pl.when(j + 1 < ch)
                def _():
                    pltpu.async_copy(g_hbm.at[src_v.at[j + 1]], nxt, nsem)

                pltpu.make_async_copy(g_hbm.at[src_v.at[j]], cur, csem).wait()
                pltpu.sync_copy(cur, acc_sp.at[dst_v.at[t]], add=True)
            return carry

        lax.fori_loop(0, nb, blk, 0)
        plsc.subcore_barrier()
        pltpu.sync_copy(acc_sp.at[pl.ds(s * zrows, zrows)],
                        out_hbm.at[c, pl.ds(s * zrows, zrows)])

    return scat_fn


def _dense_body(x_ref, w_ref, b_ref, hist_ref, g_ref):
    h = jnp.dot(x_ref[...], w_ref[...], preferred_element_type=jnp.float32)
    h = h + b_ref[...]
    nrm = jnp.sqrt(jnp.sum(h * h, axis=1, keepdims=True))
    deg = hist_ref[:, 0:1] + hist_ref[:, 1:2] + 1.0
    scale = lax.rsqrt(deg) * (_SCALE / jnp.maximum(nrm, 1e-12))
    g_ref[...] = h * scale


def _combine_body(tmp_ref, g_ref, hist_ref, o_ref):
    t = tmp_ref[0] + tmp_ref[1] + g_ref[...]
    deg = hist_ref[:, 0:1] + hist_ref[:, 1:2] + 1.0
    o_ref[...] = t * lax.rsqrt(deg)


def kernel(x, edge_index, W1, b1):
    n, d = x.shape
    e = edge_index.shape[1]
    ch = -(-e // (_NW * _K))
    ch = -(-ch // _IB) * _IB
    e_pad = _NW * ch * _K
    pad = e_pad - e
    spread = 240
    n_pad = ((n + spread + 255) // 256) * 256

    src = edge_index[0]
    dst = edge_index[1]
    pidx = jnp.arange(pad, dtype=jnp.int32)
    src2 = jnp.concatenate([src, pidx % 128]).reshape(_NW, ch, _K)
    dst2 = jnp.concatenate([dst, n + pidx % spread]).reshape(_NW, ch, _K)

    dst4 = dst2.reshape(_NW, ch // _IB, _IB, _K)

    hist = _make_degree_fn(n_pad, ch)(dst2).reshape(_NC, n_pad)
    hist_t = hist.T

    grid = n // _BLK
    g = pl.pallas_call(
        _dense_body,
        grid=(grid,),
        in_specs=[
            pl.BlockSpec((_BLK, d), lambda i: (i, 0)),
            pl.BlockSpec((d, d), lambda i: (0, 0)),
            pl.BlockSpec((1, d), lambda i: (0, 0)),
            pl.BlockSpec((_BLK, _NC), lambda i: (i, 0)),
        ],
        out_specs=pl.BlockSpec((_BLK, d), lambda i: (i, 0)),
        out_shape=jax.ShapeDtypeStruct((n, d), jnp.float32),
    )(x, W1, b1.reshape(1, d), hist_t[:n])

    tmp = _make_scatter_fn(n_pad, ch, d)(g, src2, dst4)

    out = pl.pallas_call(
        _combine_body,
        grid=(grid,),
        in_specs=[
            pl.BlockSpec((_NC, _BLK, d), lambda i: (0, i, 0)),
            pl.BlockSpec((_BLK, d), lambda i: (i, 0)),
            pl.BlockSpec((_BLK, _NC), lambda i: (i, 0)),
        ],
        out_specs=pl.BlockSpec((_BLK, d), lambda i: (i, 0)),
        out_shape=jax.ShapeDtypeStruct((n, d), jnp.float32),
    )(tmp, g, hist_t[:n])
    return out
